# Initial kernel scaffold; baseline (speedup 1.0000x reference)
import jax
import jax.numpy as jnp
from jax.experimental import pallas as pl


def kernel(context, eps, W1, b1, Wc, Wout, bout):
    raise NotImplementedError("write your pallas kernel here")


# speedup vs baseline: 2.9668x; 2.9668x over previous
import numpy as np
import jax
import jax.numpy as jnp
from jax.experimental import pallas as pl
from jax.experimental.pallas import tpu as pltpu


def _made_mask_arrays(D, H):
    m0 = np.arange(1, D + 1)
    mh = (np.arange(H) % (D - 1)) + 1
    M1 = (mh[:, None] >= m0[None, :]).astype(np.float32)
    mout = np.concatenate([m0, m0])
    Mout = (mout[:, None] > mh[None, :]).astype(np.float32)
    return M1, Mout


def _ar_body(ctx_ref, eps_ref, wct_ref, b1_ref, w1t_ref, wmu_ref, wsc_ref,
             bmu_ref, bsc_ref, z_ref, mu_ref, sc_ref, a_ref):
    BB = ctx_ref.shape[0]
    D = eps_ref.shape[-1]

    a_ref[...] = jnp.dot(ctx_ref[...], wct_ref[...],
                         preferred_element_type=jnp.float32) + b1_ref[...]

    eps_blk = eps_ref[0]
    iota = jax.lax.broadcasted_iota(jnp.int32, (1, D), 1)

    def step(i, carry):
        zac, muac, scac = carry
        oh = (iota == i).astype(jnp.float32)
        h = jnp.maximum(a_ref[...], 0.0)
        wmu_i = wmu_ref[pl.ds(i, 1), :]
        wsc_i = wsc_ref[pl.ds(i, 1), :]
        bmu_i = jnp.sum(bmu_ref[...] * oh, axis=1, keepdims=True)
        bsc_i = jnp.sum(bsc_ref[...] * oh, axis=1, keepdims=True)
        mu = jnp.sum(h * wmu_i, axis=1, keepdims=True) + bmu_i
        pre = jnp.sum(h * wsc_i, axis=1, keepdims=True) + bsc_i
        sc = jax.nn.softplus(pre)
        epsi = jnp.sum(eps_blk * oh, axis=1, keepdims=True)
        zi = mu + sc * epsi
        a_ref[...] = a_ref[...] + zi * w1t_ref[pl.ds(i, 1), :]
        return (zac + zi * oh, muac + mu * oh, scac + sc * oh)

    zeros = jnp.zeros((BB, D), jnp.float32)
    zac, muac, scac = jax.lax.fori_loop(0, D, step, (zeros, zeros, zeros))
    z_ref[0] = zac
    mu_ref[0] = muac
    sc_ref[0] = scac


def kernel(context, eps, W1, b1, Wc, Wout, bout):
    S, B, D = eps.shape
    H = W1.shape[0]
    CTX = Wc.shape[1]
    M1, Mout = _made_mask_arrays(D, H)

    W1mT = (W1 * M1).T
    WcT = Wc.T
    wmu = Wout[:D] * Mout[:D]
    wsc = Wout[D:] * Mout[D:]
    b1r = b1.reshape(1, H)
    bmu = bout[:D].reshape(1, D)
    bsc = bout[D:].reshape(1, D)

    NB = 4
    BB = B // NB

    fixed = lambda s, nb: (0, 0)
    z, mu, sc = pl.pallas_call(
        _ar_body,
        out_shape=[jax.ShapeDtypeStruct((S, B, D), jnp.float32)] * 3,
        grid=(S, NB),
        in_specs=[
            pl.BlockSpec((BB, CTX), lambda s, nb: (nb, 0)),
            pl.BlockSpec((1, BB, D), lambda s, nb: (s, nb, 0)),
            pl.BlockSpec((CTX, H), fixed),
            pl.BlockSpec((1, H), fixed),
            pl.BlockSpec((D, H), fixed),
            pl.BlockSpec((D, H), fixed),
            pl.BlockSpec((D, H), fixed),
            pl.BlockSpec((1, D), fixed),
            pl.BlockSpec((1, D), fixed),
        ],
        out_specs=[pl.BlockSpec((1, BB, D), lambda s, nb: (s, nb, 0))] * 3,
        scratch_shapes=[pltpu.VMEM((BB, H), jnp.float32)],
        compiler_params=pltpu.CompilerParams(
            dimension_semantics=("parallel", "arbitrary"),
            vmem_limit_bytes=48 * 1024 * 1024,
        ),
        name="made_ar_sample",
    )(context, eps, WcT, b1r, W1mT, wmu, wsc, bmu, bsc)
    return z, mu, sc


# speedup vs baseline: 3.0066x; 1.0134x over previous
import numpy as np
import jax
import jax.numpy as jnp
from jax.experimental import pallas as pl
from jax.experimental.pallas import tpu as pltpu


def _made_mask_arrays(D, H):
    m0 = np.arange(1, D + 1)
    mh = (np.arange(H) % (D - 1)) + 1
    M1 = (mh[:, None] >= m0[None, :]).astype(np.float32)
    mout = np.concatenate([m0, m0])
    Mout = (mout[:, None] > mh[None, :]).astype(np.float32)
    return M1, Mout


def _ar_body(ctx_ref, eps_ref, wct_ref, b1_ref, w1t_ref, wmu_ref, wsc_ref,
             bmu_ref, bsc_ref, z_ref, mu_ref, sc_ref, a_ref):
    BB = ctx_ref.shape[0]
    D = eps_ref.shape[-1]

    a_ref[...] = jnp.dot(ctx_ref[...], wct_ref[...],
                         preferred_element_type=jnp.float32) + b1_ref[...]

    eps_blk = eps_ref[0]
    iota = jax.lax.broadcasted_iota(jnp.int32, (1, D), 1)

    def step(i, carry):
        zac, muac, scac = carry
        oh = (iota == i).astype(jnp.float32)
        h = jnp.maximum(a_ref[...], 0.0)
        wmu_i = wmu_ref[pl.ds(i, 1), :]
        wsc_i = wsc_ref[pl.ds(i, 1), :]
        bmu_i = jnp.sum(bmu_ref[...] * oh, axis=1, keepdims=True)
        bsc_i = jnp.sum(bsc_ref[...] * oh, axis=1, keepdims=True)
        mu = jnp.sum(h * wmu_i, axis=1, keepdims=True) + bmu_i
        pre = jnp.sum(h * wsc_i, axis=1, keepdims=True) + bsc_i
        sc = jax.nn.softplus(pre)
        epsi = jnp.sum(eps_blk * oh, axis=1, keepdims=True)
        zi = mu + sc * epsi
        a_ref[...] = a_ref[...] + zi * w1t_ref[pl.ds(i, 1), :]
        return (zac + zi * oh, muac + mu * oh, scac + sc * oh)

    zeros = jnp.zeros((BB, D), jnp.float32)
    zac, muac, scac = jax.lax.fori_loop(0, D, step, (zeros, zeros, zeros))
    z_ref[0] = zac
    mu_ref[0] = muac
    sc_ref[0] = scac


def kernel(context, eps, W1, b1, Wc, Wout, bout):
    S, B, D = eps.shape
    H = W1.shape[0]
    CTX = Wc.shape[1]
    M1, Mout = _made_mask_arrays(D, H)

    W1mT = (W1 * M1).T
    WcT = Wc.T
    wmu = Wout[:D] * Mout[:D]
    wsc = Wout[D:] * Mout[D:]
    b1r = b1.reshape(1, H)
    bmu = bout[:D].reshape(1, D)
    bsc = bout[D:].reshape(1, D)

    NB = 1
    BB = B // NB

    fixed = lambda s, nb: (0, 0)
    z, mu, sc = pl.pallas_call(
        _ar_body,
        out_shape=[jax.ShapeDtypeStruct((S, B, D), jnp.float32)] * 3,
        grid=(S, NB),
        in_specs=[
            pl.BlockSpec((BB, CTX), lambda s, nb: (nb, 0)),
            pl.BlockSpec((1, BB, D), lambda s, nb: (s, nb, 0)),
            pl.BlockSpec((CTX, H), fixed),
            pl.BlockSpec((1, H), fixed),
            pl.BlockSpec((D, H), fixed),
            pl.BlockSpec((D, H), fixed),
            pl.BlockSpec((D, H), fixed),
            pl.BlockSpec((1, D), fixed),
            pl.BlockSpec((1, D), fixed),
        ],
        out_specs=[pl.BlockSpec((1, BB, D), lambda s, nb: (s, nb, 0))] * 3,
        scratch_shapes=[pltpu.VMEM((BB, H), jnp.float32)],
        compiler_params=pltpu.CompilerParams(
            dimension_semantics=("parallel", "arbitrary"),
            vmem_limit_bytes=48 * 1024 * 1024,
        ),
        name="made_ar_sample",
    )(context, eps, WcT, b1r, W1mT, wmu, wsc, bmu, bsc)
    return z, mu, sc


# speedup vs baseline: 3.8378x; 1.2765x over previous
import numpy as np
import jax
import jax.numpy as jnp
from jax.experimental import pallas as pl
from jax.experimental.pallas import tpu as pltpu

_LANE = 128


def _made_degrees(D, H):
    m0 = np.arange(1, D + 1)
    mh = (np.arange(H) % (D - 1)) + 1
    return m0, mh


def _ar_body(ctx_ref, eps_ref, wct_ref, b1_ref, w1t_ref, wmu_ref, wsc_ref,
             bmu_ref, bsc_ref, z_ref, mu_ref, sc_ref, a_ref, *, group_bounds):
    BB = ctx_ref.shape[0]
    D = eps_ref.shape[-1]
    H = a_ref.shape[-1]

    a_ref[...] = jnp.dot(ctx_ref[...], wct_ref[...],
                         preferred_element_type=jnp.float32) + b1_ref[...]

    eps_blk = eps_ref[0]
    iota = jax.lax.broadcasted_iota(jnp.int32, (1, D), 1)

    def make_step(pw, s0):
        def step(i, carry):
            zac, muac, scac = carry
            oh = (iota == i).astype(jnp.float32)
            h = jnp.maximum(a_ref[:, :pw], 0.0)
            wmu_i = wmu_ref[pl.ds(i, 1), :][:, :pw]
            wsc_i = wsc_ref[pl.ds(i, 1), :][:, :pw]
            bmu_i = jnp.sum(bmu_ref[...] * oh, axis=1, keepdims=True)
            bsc_i = jnp.sum(bsc_ref[...] * oh, axis=1, keepdims=True)
            mu = jnp.sum(h * wmu_i, axis=1, keepdims=True) + bmu_i
            pre = jnp.sum(h * wsc_i, axis=1, keepdims=True) + bsc_i
            sc = jax.nn.softplus(pre)
            epsi = jnp.sum(eps_blk * oh, axis=1, keepdims=True)
            zi = mu + sc * epsi
            a_ref[:, s0:] = a_ref[:, s0:] + zi * w1t_ref[pl.ds(i, 1), :][:, s0:]
            return (zac + zi * oh, muac + mu * oh, scac + sc * oh)
        return step

    zeros = jnp.zeros((BB, D), jnp.float32)
    carry = (zeros, zeros, zeros)
    for (i0, i1, pw, s0) in group_bounds:
        carry = jax.lax.fori_loop(i0, i1, make_step(pw, s0), carry)
    zac, muac, scac = carry
    z_ref[0] = zac
    mu_ref[0] = muac
    sc_ref[0] = scac


def kernel(context, eps, W1, b1, Wc, Wout, bout):
    S, B, D = eps.shape
    H = W1.shape[0]
    CTX = Wc.shape[1]
    m0, mh = _made_degrees(D, H)

    perm = np.argsort(mh, kind="stable")
    mh_s = mh[perm]
    M1 = jnp.asarray((mh_s[:, None] >= m0[None, :]).astype(np.float32))
    Mout = jnp.asarray((m0[:, None] > mh_s[None, :]).astype(np.float32))
    perm_j = jnp.asarray(perm)

    W1p = W1[perm_j]
    Wcp = Wc[perm_j]
    b1p = b1[perm_j]
    Woutp = Wout[:, perm_j]

    W1mT = (W1p * M1).T
    WcT = Wcp.T
    wmu = Woutp[:D] * Mout
    wsc = Woutp[D:] * Mout
    b1r = b1p.reshape(1, H)
    bmu = bout[:D].reshape(1, D)
    bsc = bout[D:].reshape(1, D)

    n = np.cumsum(np.bincount(mh_s, minlength=D + 1))[:D]
    GROUP = 8
    group_bounds = []
    for i0 in range(0, D, GROUP):
        i1 = min(i0 + GROUP, D)
        pw = int(min(max(-(-n[i1 - 1] // _LANE), 1), H // _LANE)) * _LANE
        s0 = int(min(n[i0] // _LANE, H // _LANE - 1)) * _LANE
        group_bounds.append((i0, i1, pw, s0))

    NB = 1
    BB = B // NB

    import functools
    body = functools.partial(_ar_body, group_bounds=tuple(group_bounds))

    fixed = lambda s, nb: (0, 0)
    z, mu, sc = pl.pallas_call(
        body,
        out_shape=[jax.ShapeDtypeStruct((S, B, D), jnp.float32)] * 3,
        grid=(S, NB),
        in_specs=[
            pl.BlockSpec((BB, CTX), lambda s, nb: (nb, 0)),
            pl.BlockSpec((1, BB, D), lambda s, nb: (s, nb, 0)),
            pl.BlockSpec((CTX, H), fixed),
            pl.BlockSpec((1, H), fixed),
            pl.BlockSpec((D, H), fixed),
            pl.BlockSpec((D, H), fixed),
            pl.BlockSpec((D, H), fixed),
            pl.BlockSpec((1, D), fixed),
            pl.BlockSpec((1, D), fixed),
        ],
        out_specs=[pl.BlockSpec((1, BB, D), lambda s, nb: (s, nb, 0))] * 3,
        scratch_shapes=[pltpu.VMEM((BB, H), jnp.float32)],
        compiler_params=pltpu.CompilerParams(
            dimension_semantics=("parallel", "arbitrary"),
            vmem_limit_bytes=48 * 1024 * 1024,
        ),
        name="made_ar_sample",
    )(context, eps, WcT, b1r, W1mT, wmu, wsc, bmu, bsc)
    return z, mu, sc


# speedup vs baseline: 4.1290x; 1.0759x over previous
import numpy as np
import jax
import jax.numpy as jnp
from jax.experimental import pallas as pl
from jax.experimental.pallas import tpu as pltpu

_LANE = 128
_GROUP = 8


def _made_degrees(D, H):
    m0 = np.arange(1, D + 1)
    mh = (np.arange(H) % (D - 1)) + 1
    return m0, mh


def _ar_body(ctx_ref, eps_ref, wct_ref, b1_ref, w1t_ref, wmu_ref, wsc_ref,
             wall_ref, wpart_ref, bmu_ref, bsc_ref,
             z_ref, mu_ref, sc_ref, a_ref, h_ref):
    BB = ctx_ref.shape[0]
    D = eps_ref.shape[-1]
    H = a_ref.shape[-1]
    NG = D // _GROUP

    a_ref[...] = jnp.dot(ctx_ref[...], wct_ref[...],
                         preferred_element_type=jnp.float32) + b1_ref[...]

    eps_blk = eps_ref[0]
    iota = jax.lax.broadcasted_iota(jnp.int32, (1, D), 1)
    iota16 = jax.lax.broadcasted_iota(jnp.int32, (1, 2 * _GROUP), 1)
    cdims = (((1,), (1,)), ((), ()))

    zac = jnp.zeros((BB, D), jnp.float32)
    for g in range(NG):
        i0 = g * _GROUP
        c0 = g * _LANE
        c1 = min((g + 2) * _LANE, H)

        lz0, lz1 = (g + 1) * _LANE, min((g + 2) * _LANE, H)
        if g >= 1 and lz0 < H:
            a_ref[:, lz0:lz1] = a_ref[:, lz0:lz1] + jax.lax.dot_general(
                zac, w1t_ref[:, lz0:lz1], (((1,), (0,)), ((), ())),
                preferred_element_type=jnp.float32)

        if g > 0:
            kf = g * _LANE
            part = jax.lax.dot_general(
                h_ref[:, :kf], wpart_ref[g][:, :kf], cdims,
                preferred_element_type=jnp.float32)
        else:
            part = None

        def step(i, zac, part=part, c0=c0, c1=c1, i0=i0):
            oh = (iota == i).astype(jnp.float32)
            win = a_ref[:, c0:c1]
            h = jnp.maximum(win, 0.0)
            wmu_row = wmu_ref[pl.ds(i, 1), :]
            wsc_row = wsc_ref[pl.ds(i, 1), :]
            w1_row = w1t_ref[pl.ds(i, 1), :]
            mu = jnp.sum(h * wmu_row[:, c0:c1], axis=1, keepdims=True)
            pre = jnp.sum(h * wsc_row[:, c0:c1], axis=1, keepdims=True)
            if part is not None:
                j = i - i0
                mu = mu + jnp.sum(part * (iota16 == j).astype(jnp.float32),
                                  axis=1, keepdims=True)
                pre = pre + jnp.sum(part * (iota16 == j + _GROUP).astype(jnp.float32),
                                    axis=1, keepdims=True)
            mu = mu + jnp.sum(bmu_ref[...] * oh, axis=1, keepdims=True)
            pre = pre + jnp.sum(bsc_ref[...] * oh, axis=1, keepdims=True)
            sc = jax.nn.softplus(pre)
            epsi = jnp.sum(eps_blk * oh, axis=1, keepdims=True)
            zi = mu + sc * epsi
            a_ref[:, c0:c1] = win + zi * w1_row[:, c0:c1]
            return zac + zi * oh

        zac = jax.lax.fori_loop(i0, i0 + _GROUP, step, zac)

        f1 = min(c0 + _LANE, H)
        h_ref[:, c0:f1] = jnp.maximum(a_ref[:, c0:f1], 0.0)

    z_ref[0] = zac
    out = jax.lax.dot_general(h_ref[...], wall_ref[...], cdims,
                              preferred_element_type=jnp.float32)
    mu_ref[0] = out[:, :D] + bmu_ref[...]
    sc_ref[0] = jax.nn.softplus(out[:, D:] + bsc_ref[...])


def kernel(context, eps, W1, b1, Wc, Wout, bout):
    S, B, D = eps.shape
    H = W1.shape[0]
    CTX = Wc.shape[1]
    m0, mh = _made_degrees(D, H)

    perm = np.argsort(mh, kind="stable")
    mh_s = mh[perm]
    M1 = jnp.asarray((mh_s[:, None] >= m0[None, :]).astype(np.float32))
    Mout = jnp.asarray((m0[:, None] > mh_s[None, :]).astype(np.float32))
    perm_j = jnp.asarray(perm)

    W1p = W1[perm_j]
    Wcp = Wc[perm_j]
    b1p = b1[perm_j]
    Woutp = Wout[:, perm_j]

    W1mT = (W1p * M1).T
    WcT = Wcp.T
    wmu = Woutp[:D] * Mout
    wsc = Woutp[D:] * Mout
    wall = jnp.concatenate([wmu, wsc], axis=0)
    b1r = b1p.reshape(1, H)
    bmu = bout[:D].reshape(1, D)
    bsc = bout[D:].reshape(1, D)

    NG = D // _GROUP
    wpart = jnp.stack([
        jnp.concatenate([wmu[g * _GROUP:(g + 1) * _GROUP],
                         wsc[g * _GROUP:(g + 1) * _GROUP]], axis=0)
        for g in range(NG)])

    NB = 1
    BB = B // NB

    fixed = lambda s, nb: (0, 0)
    fixed3 = lambda s, nb: (0, 0, 0)
    z, mu, sc = pl.pallas_call(
        _ar_body,
        out_shape=[jax.ShapeDtypeStruct((S, B, D), jnp.float32)] * 3,
        grid=(S, NB),
        in_specs=[
            pl.BlockSpec((BB, CTX), lambda s, nb: (nb, 0)),
            pl.BlockSpec((1, BB, D), lambda s, nb: (s, nb, 0)),
            pl.BlockSpec((CTX, H), fixed),
            pl.BlockSpec((1, H), fixed),
            pl.BlockSpec((D, H), fixed),
            pl.BlockSpec((D, H), fixed),
            pl.BlockSpec((D, H), fixed),
            pl.BlockSpec((2 * D, H), fixed),
            pl.BlockSpec((NG, 2 * _GROUP, H), fixed3),
            pl.BlockSpec((1, D), fixed),
            pl.BlockSpec((1, D), fixed),
        ],
        out_specs=[pl.BlockSpec((1, BB, D), lambda s, nb: (s, nb, 0))] * 3,
        scratch_shapes=[pltpu.VMEM((BB, H), jnp.float32),
                        pltpu.VMEM((BB, H), jnp.float32)],
        compiler_params=pltpu.CompilerParams(
            dimension_semantics=("parallel", "arbitrary"),
            vmem_limit_bytes=48 * 1024 * 1024,
        ),
        name="made_ar_sample",
    )(context, eps, WcT, b1r, W1mT, wmu, wsc, wall, wpart, bmu, bsc)
    return z, mu, sc


# speedup vs baseline: 4.2751x; 1.0354x over previous
import numpy as np
import jax
import jax.numpy as jnp
from jax.experimental import pallas as pl
from jax.experimental.pallas import tpu as pltpu

_LANE = 128
_GROUP = 8


def _made_degrees(D, H):
    m0 = np.arange(1, D + 1)
    mh = (np.arange(H) % (D - 1)) + 1
    return m0, mh


def _ar_body(ctx_ref, eps_ref, wct_ref, b1_ref, w1t_ref, wmu_ref, wsc_ref,
             wall_ref, wpart_ref, bmu_ref, bsc_ref,
             z_ref, mu_ref, sc_ref, a_ref, h_ref, zs_ref, part_ref):
    BB = ctx_ref.shape[0]
    D = eps_ref.shape[-1]
    H = a_ref.shape[-1]
    NG = D // _GROUP

    a_ref[...] = jnp.dot(ctx_ref[...], wct_ref[...],
                         preferred_element_type=jnp.float32) + b1_ref[...]

    iota = jax.lax.broadcasted_iota(jnp.int32, (1, D), 1)
    iota16 = jax.lax.broadcasted_iota(jnp.int32, (1, 2 * _GROUP), 1)
    cdims = (((1,), (1,)), ((), ()))

    zs_ref[...] = jnp.zeros((BB, D), jnp.float32)
    for g in range(NG):
        i0 = g * _GROUP
        c0 = g * _LANE
        c1 = min((g + 2) * _LANE, H)

        lz0, lz1 = (g + 1) * _LANE, min((g + 2) * _LANE, H)
        if g >= 1 and lz0 < H:
            a_ref[:, lz0:lz1] = a_ref[:, lz0:lz1] + jax.lax.dot_general(
                zs_ref[...], w1t_ref[:, lz0:lz1], (((1,), (0,)), ((), ())),
                preferred_element_type=jnp.float32)

        has_part = g > 0
        if has_part:
            kf = g * _LANE
            part_ref[...] = jax.lax.dot_general(
                h_ref[:, :kf], wpart_ref[g][:, :kf], cdims,
                preferred_element_type=jnp.float32)

        def step(i, _, has_part=has_part, c0=c0, c1=c1, i0=i0):
            oh = (iota == i).astype(jnp.float32)
            win = a_ref[:, c0:c1]
            h = jnp.maximum(win, 0.0)
            wmu_row = wmu_ref[pl.ds(i, 1), :]
            wsc_row = wsc_ref[pl.ds(i, 1), :]
            w1_row = w1t_ref[pl.ds(i, 1), :]
            mu = jnp.sum(h * wmu_row[:, c0:c1], axis=1, keepdims=True)
            pre = jnp.sum(h * wsc_row[:, c0:c1], axis=1, keepdims=True)
            if has_part:
                j = i - i0
                part = part_ref[...]
                mu = mu + jnp.sum(part * (iota16 == j).astype(jnp.float32),
                                  axis=1, keepdims=True)
                pre = pre + jnp.sum(part * (iota16 == j + _GROUP).astype(jnp.float32),
                                    axis=1, keepdims=True)
            mu = mu + jnp.sum(bmu_ref[...] * oh, axis=1, keepdims=True)
            pre = pre + jnp.sum(bsc_ref[...] * oh, axis=1, keepdims=True)
            sc = jax.nn.softplus(pre)
            epsi = jnp.sum(eps_ref[0] * oh, axis=1, keepdims=True)
            zi = mu + sc * epsi
            a_ref[:, c0:c1] = win + zi * w1_row[:, c0:c1]
            zs_ref[...] = zs_ref[...] + zi * oh
            return 0

        jax.lax.fori_loop(i0, i0 + _GROUP, step, 0)

        f1 = min(c0 + _LANE, H)
        h_ref[:, c0:f1] = jnp.maximum(a_ref[:, c0:f1], 0.0)

    z_ref[0] = zs_ref[...]
    out = jax.lax.dot_general(h_ref[...], wall_ref[...], cdims,
                              preferred_element_type=jnp.float32)
    mu_ref[0] = out[:, :D] + bmu_ref[...]
    sc_ref[0] = jax.nn.softplus(out[:, D:] + bsc_ref[...])


def kernel(context, eps, W1, b1, Wc, Wout, bout):
    S, B, D = eps.shape
    H = W1.shape[0]
    CTX = Wc.shape[1]
    m0, mh = _made_degrees(D, H)

    perm = np.argsort(mh, kind="stable")
    mh_s = mh[perm]
    M1 = jnp.asarray((mh_s[:, None] >= m0[None, :]).astype(np.float32))
    Mout = jnp.asarray((m0[:, None] > mh_s[None, :]).astype(np.float32))
    perm_j = jnp.asarray(perm)

    W1p = W1[perm_j]
    Wcp = Wc[perm_j]
    b1p = b1[perm_j]
    Woutp = Wout[:, perm_j]

    W1mT = (W1p * M1).T
    WcT = Wcp.T
    wmu = Woutp[:D] * Mout
    wsc = Woutp[D:] * Mout
    wall = jnp.concatenate([wmu, wsc], axis=0)
    b1r = b1p.reshape(1, H)
    bmu = bout[:D].reshape(1, D)
    bsc = bout[D:].reshape(1, D)

    NG = D // _GROUP
    wpart = jnp.stack([
        jnp.concatenate([wmu[g * _GROUP:(g + 1) * _GROUP],
                         wsc[g * _GROUP:(g + 1) * _GROUP]], axis=0)
        for g in range(NG)])

    NB = 1
    BB = B // NB

    fixed = lambda s, nb: (0, 0)
    fixed3 = lambda s, nb: (0, 0, 0)
    z, mu, sc = pl.pallas_call(
        _ar_body,
        out_shape=[jax.ShapeDtypeStruct((S, B, D), jnp.float32)] * 3,
        grid=(S, NB),
        in_specs=[
            pl.BlockSpec((BB, CTX), lambda s, nb: (nb, 0)),
            pl.BlockSpec((1, BB, D), lambda s, nb: (s, nb, 0)),
            pl.BlockSpec((CTX, H), fixed),
            pl.BlockSpec((1, H), fixed),
            pl.BlockSpec((D, H), fixed),
            pl.BlockSpec((D, H), fixed),
            pl.BlockSpec((D, H), fixed),
            pl.BlockSpec((2 * D, H), fixed),
            pl.BlockSpec((NG, 2 * _GROUP, H), fixed3),
            pl.BlockSpec((1, D), fixed),
            pl.BlockSpec((1, D), fixed),
        ],
        out_specs=[pl.BlockSpec((1, BB, D), lambda s, nb: (s, nb, 0))] * 3,
        scratch_shapes=[pltpu.VMEM((BB, H), jnp.float32),
                        pltpu.VMEM((BB, H), jnp.float32),
                        pltpu.VMEM((BB, D), jnp.float32),
                        pltpu.VMEM((BB, 2 * _GROUP), jnp.float32)],
        compiler_params=pltpu.CompilerParams(
            dimension_semantics=("parallel", "arbitrary"),
            vmem_limit_bytes=48 * 1024 * 1024,
        ),
        name="made_ar_sample",
    )(context, eps, WcT, b1r, W1mT, wmu, wsc, wall, wpart, bmu, bsc)
    return z, mu, sc


# speedup vs baseline: 4.7623x; 1.1140x over previous
import numpy as np
import jax
import jax.numpy as jnp
from jax.experimental import pallas as pl
from jax.experimental.pallas import tpu as pltpu

_LANE = 128
_GROUP = 8
_RSPLIT = 4


def _made_degrees(D, H):
    m0 = np.arange(1, D + 1)
    mh = (np.arange(H) % (D - 1)) + 1
    return m0, mh


def _ar_body(ctx_ref, eps_ref, wct_ref, b1_ref, w1t_ref, wmu_ref, wsc_ref,
             wall_ref, wpart_ref, bmu_ref, bsc_ref,
             z_ref, mu_ref, sc_ref, a_ref, h_ref, zs_ref, part_ref):
    BB = ctx_ref.shape[0]
    D = eps_ref.shape[-1]
    H = a_ref.shape[-1]
    NG = D // _GROUP

    a_ref[...] = jnp.dot(ctx_ref[...], wct_ref[...],
                         preferred_element_type=jnp.float32) + b1_ref[...]

    iota = jax.lax.broadcasted_iota(jnp.int32, (1, D), 1)
    iota16 = jax.lax.broadcasted_iota(jnp.int32, (1, 2 * _GROUP), 1)
    cdims = (((1,), (1,)), ((), ()))

    zs_ref[...] = jnp.zeros((BB, D), jnp.float32)
    for g in range(NG):
        i0 = g * _GROUP
        c0 = g * _LANE
        c1 = min((g + 2) * _LANE, H)

        lz0, lz1 = (g + 1) * _LANE, min((g + 2) * _LANE, H)
        if g >= 1 and lz0 < H:
            a_ref[:, lz0:lz1] = a_ref[:, lz0:lz1] + jax.lax.dot_general(
                zs_ref[...], w1t_ref[:, lz0:lz1], (((1,), (0,)), ((), ())),
                preferred_element_type=jnp.float32)

        has_part = g > 0
        if has_part:
            kf = g * _LANE
            part_ref[...] = jax.lax.dot_general(
                h_ref[:, :kf], wpart_ref[g][:, :kf], cdims,
                preferred_element_type=jnp.float32)

        def step(i, _, has_part=has_part, c0=c0, c1=c1, i0=i0):
            oh = (iota == i).astype(jnp.float32)
            wmu_row = wmu_ref[pl.ds(i, 1), :]
            wsc_row = wsc_ref[pl.ds(i, 1), :]
            w1_row = w1t_ref[pl.ds(i, 1), :]
            j = i - i0
            mpm = (iota16 == j).astype(jnp.float32)
            mpp = (iota16 == j + _GROUP).astype(jnp.float32)
            bmu_i = jnp.sum(bmu_ref[...] * oh, axis=1, keepdims=True)
            bsc_i = jnp.sum(bsc_ref[...] * oh, axis=1, keepdims=True)
            RB = BB // _RSPLIT
            for r0 in range(0, BB, RB):
                r1 = r0 + RB
                win = a_ref[r0:r1, c0:c1]
                h = jnp.maximum(win, 0.0)
                mu = jnp.sum(h * wmu_row[:, c0:c1], axis=1, keepdims=True)
                pre = jnp.sum(h * wsc_row[:, c0:c1], axis=1, keepdims=True)
                if has_part:
                    part = part_ref[r0:r1, :]
                    mu = mu + jnp.sum(part * mpm, axis=1, keepdims=True)
                    pre = pre + jnp.sum(part * mpp, axis=1, keepdims=True)
                mu = mu + bmu_i
                pre = pre + bsc_i
                sc = jax.nn.softplus(pre)
                epsi = jnp.sum(eps_ref[0, r0:r1, :] * oh, axis=1, keepdims=True)
                zi = mu + sc * epsi
                a_ref[r0:r1, c0:c1] = win + zi * w1_row[:, c0:c1]
                zs_ref[r0:r1, :] = zs_ref[r0:r1, :] + zi * oh
            return 0

        jax.lax.fori_loop(i0, i0 + _GROUP, step, 0)

        f1 = min(c0 + _LANE, H)
        h_ref[:, c0:f1] = jnp.maximum(a_ref[:, c0:f1], 0.0)

    z_ref[0] = zs_ref[...]
    out = jax.lax.dot_general(h_ref[...], wall_ref[...], cdims,
                              preferred_element_type=jnp.float32)
    mu_ref[0] = out[:, :D] + bmu_ref[...]
    sc_ref[0] = jax.nn.softplus(out[:, D:] + bsc_ref[...])


def kernel(context, eps, W1, b1, Wc, Wout, bout):
    S, B, D = eps.shape
    H = W1.shape[0]
    CTX = Wc.shape[1]
    m0, mh = _made_degrees(D, H)

    perm = np.argsort(mh, kind="stable")
    mh_s = mh[perm]
    M1 = jnp.asarray((mh_s[:, None] >= m0[None, :]).astype(np.float32))
    Mout = jnp.asarray((m0[:, None] > mh_s[None, :]).astype(np.float32))
    perm_j = jnp.asarray(perm)

    W1p = W1[perm_j]
    Wcp = Wc[perm_j]
    b1p = b1[perm_j]
    Woutp = Wout[:, perm_j]

    W1mT = (W1p * M1).T
    WcT = Wcp.T
    wmu = Woutp[:D] * Mout
    wsc = Woutp[D:] * Mout
    wall = jnp.concatenate([wmu, wsc], axis=0)
    b1r = b1p.reshape(1, H)
    bmu = bout[:D].reshape(1, D)
    bsc = bout[D:].reshape(1, D)

    NG = D // _GROUP
    wpart = jnp.stack([
        jnp.concatenate([wmu[g * _GROUP:(g + 1) * _GROUP],
                         wsc[g * _GROUP:(g + 1) * _GROUP]], axis=0)
        for g in range(NG)])

    NB = 1
    BB = B // NB

    fixed = lambda s, nb: (0, 0)
    fixed3 = lambda s, nb: (0, 0, 0)
    z, mu, sc = pl.pallas_call(
        _ar_body,
        out_shape=[jax.ShapeDtypeStruct((S, B, D), jnp.float32)] * 3,
        grid=(S, NB),
        in_specs=[
            pl.BlockSpec((BB, CTX), lambda s, nb: (nb, 0)),
            pl.BlockSpec((1, BB, D), lambda s, nb: (s, nb, 0)),
            pl.BlockSpec((CTX, H), fixed),
            pl.BlockSpec((1, H), fixed),
            pl.BlockSpec((D, H), fixed),
            pl.BlockSpec((D, H), fixed),
            pl.BlockSpec((D, H), fixed),
            pl.BlockSpec((2 * D, H), fixed),
            pl.BlockSpec((NG, 2 * _GROUP, H), fixed3),
            pl.BlockSpec((1, D), fixed),
            pl.BlockSpec((1, D), fixed),
        ],
        out_specs=[pl.BlockSpec((1, BB, D), lambda s, nb: (s, nb, 0))] * 3,
        scratch_shapes=[pltpu.VMEM((BB, H), jnp.float32),
                        pltpu.VMEM((BB, H), jnp.float32),
                        pltpu.VMEM((BB, D), jnp.float32),
                        pltpu.VMEM((BB, 2 * _GROUP), jnp.float32)],
        compiler_params=pltpu.CompilerParams(
            dimension_semantics=("parallel", "arbitrary"),
            vmem_limit_bytes=48 * 1024 * 1024,
        ),
        name="made_ar_sample",
    )(context, eps, WcT, b1r, W1mT, wmu, wsc, wall, wpart, bmu, bsc)
    return z, mu, sc


# speedup vs baseline: 5.3910x; 1.1320x over previous
import numpy as np
import jax
import jax.numpy as jnp
from jax.experimental import pallas as pl
from jax.experimental.pallas import tpu as pltpu

_LANE = 128
_GROUP = 8
_RSPLIT = 8


def _made_degrees(D, H):
    m0 = np.arange(1, D + 1)
    mh = (np.arange(H) % (D - 1)) + 1
    return m0, mh


def _ar_body(ctx_ref, eps_ref, wct_ref, b1_ref, w1t_ref, wmu_ref, wsc_ref,
             wall_ref, wpart_ref, bmu_ref, bsc_ref,
             z_ref, mu_ref, sc_ref, a_ref, h_ref, zs_ref, part_ref):
    BB = ctx_ref.shape[0]
    D = eps_ref.shape[-1]
    H = a_ref.shape[-1]
    NG = D // _GROUP

    a_ref[...] = jnp.dot(ctx_ref[...], wct_ref[...],
                         preferred_element_type=jnp.float32) + b1_ref[...]

    iota = jax.lax.broadcasted_iota(jnp.int32, (1, D), 1)
    iota16 = jax.lax.broadcasted_iota(jnp.int32, (1, 2 * _GROUP), 1)
    cdims = (((1,), (1,)), ((), ()))

    zs_ref[...] = jnp.zeros((BB, D), jnp.float32)
    for g in range(NG):
        i0 = g * _GROUP
        c0 = g * _LANE
        c1 = min((g + 2) * _LANE, H)

        lz0, lz1 = (g + 1) * _LANE, min((g + 2) * _LANE, H)
        if g >= 1 and lz0 < H:
            a_ref[:, lz0:lz1] = a_ref[:, lz0:lz1] + jax.lax.dot_general(
                zs_ref[...], w1t_ref[:, lz0:lz1], (((1,), (0,)), ((), ())),
                preferred_element_type=jnp.float32)

        has_part = g > 0
        if has_part:
            kf = g * _LANE
            part_ref[...] = jax.lax.dot_general(
                h_ref[:, :kf], wpart_ref[g][:, :kf], cdims,
                preferred_element_type=jnp.float32)

        def step(i, _, has_part=has_part, c0=c0, c1=c1, i0=i0):
            oh = (iota == i).astype(jnp.float32)
            wmu_row = wmu_ref[pl.ds(i, 1), :]
            wsc_row = wsc_ref[pl.ds(i, 1), :]
            w1_row = w1t_ref[pl.ds(i, 1), :]
            j = i - i0
            mpm = (iota16 == j).astype(jnp.float32)
            mpp = (iota16 == j + _GROUP).astype(jnp.float32)
            bmu_i = jnp.sum(bmu_ref[...] * oh, axis=1, keepdims=True)
            bsc_i = jnp.sum(bsc_ref[...] * oh, axis=1, keepdims=True)
            RB = BB // _RSPLIT
            for r0 in range(0, BB, RB):
                r1 = r0 + RB
                win = a_ref[r0:r1, c0:c1]
                h = jnp.maximum(win, 0.0)
                mu = jnp.sum(h * wmu_row[:, c0:c1], axis=1, keepdims=True)
                pre = jnp.sum(h * wsc_row[:, c0:c1], axis=1, keepdims=True)
                if has_part:
                    part = part_ref[r0:r1, :]
                    mu = mu + jnp.sum(part * mpm, axis=1, keepdims=True)
                    pre = pre + jnp.sum(part * mpp, axis=1, keepdims=True)
                mu = mu + bmu_i
                pre = pre + bsc_i
                sc = jax.nn.softplus(pre)
                epsi = jnp.sum(eps_ref[0, r0:r1, :] * oh, axis=1, keepdims=True)
                zi = mu + sc * epsi
                a_ref[r0:r1, c0:c1] = win + zi * w1_row[:, c0:c1]
                zs_ref[r0:r1, :] = zs_ref[r0:r1, :] + zi * oh
            return 0

        jax.lax.fori_loop(i0, i0 + _GROUP, step, 0)

        f1 = min(c0 + _LANE, H)
        h_ref[:, c0:f1] = jnp.maximum(a_ref[:, c0:f1], 0.0)

    z_ref[0] = zs_ref[...]
    out = jax.lax.dot_general(h_ref[...], wall_ref[...], cdims,
                              preferred_element_type=jnp.float32)
    mu_ref[0] = out[:, :D] + bmu_ref[...]
    sc_ref[0] = jax.nn.softplus(out[:, D:] + bsc_ref[...])


def kernel(context, eps, W1, b1, Wc, Wout, bout):
    S, B, D = eps.shape
    H = W1.shape[0]
    CTX = Wc.shape[1]
    m0, mh = _made_degrees(D, H)

    perm = np.argsort(mh, kind="stable")
    mh_s = mh[perm]
    M1 = jnp.asarray((mh_s[:, None] >= m0[None, :]).astype(np.float32))
    Mout = jnp.asarray((m0[:, None] > mh_s[None, :]).astype(np.float32))
    perm_j = jnp.asarray(perm)

    W1p = W1[perm_j]
    Wcp = Wc[perm_j]
    b1p = b1[perm_j]
    Woutp = Wout[:, perm_j]

    W1mT = (W1p * M1).T
    WcT = Wcp.T
    wmu = Woutp[:D] * Mout
    wsc = Woutp[D:] * Mout
    wall = jnp.concatenate([wmu, wsc], axis=0)
    b1r = b1p.reshape(1, H)
    bmu = bout[:D].reshape(1, D)
    bsc = bout[D:].reshape(1, D)

    NG = D // _GROUP
    wpart = jnp.stack([
        jnp.concatenate([wmu[g * _GROUP:(g + 1) * _GROUP],
                         wsc[g * _GROUP:(g + 1) * _GROUP]], axis=0)
        for g in range(NG)])

    NB = 1
    BB = B // NB

    fixed = lambda s, nb: (0, 0)
    fixed3 = lambda s, nb: (0, 0, 0)
    z, mu, sc = pl.pallas_call(
        _ar_body,
        out_shape=[jax.ShapeDtypeStruct((S, B, D), jnp.float32)] * 3,
        grid=(S, NB),
        in_specs=[
            pl.BlockSpec((BB, CTX), lambda s, nb: (nb, 0)),
            pl.BlockSpec((1, BB, D), lambda s, nb: (s, nb, 0)),
            pl.BlockSpec((CTX, H), fixed),
            pl.BlockSpec((1, H), fixed),
            pl.BlockSpec((D, H), fixed),
            pl.BlockSpec((D, H), fixed),
            pl.BlockSpec((D, H), fixed),
            pl.BlockSpec((2 * D, H), fixed),
            pl.BlockSpec((NG, 2 * _GROUP, H), fixed3),
            pl.BlockSpec((1, D), fixed),
            pl.BlockSpec((1, D), fixed),
        ],
        out_specs=[pl.BlockSpec((1, BB, D), lambda s, nb: (s, nb, 0))] * 3,
        scratch_shapes=[pltpu.VMEM((BB, H), jnp.float32),
                        pltpu.VMEM((BB, H), jnp.float32),
                        pltpu.VMEM((BB, D), jnp.float32),
                        pltpu.VMEM((BB, 2 * _GROUP), jnp.float32)],
        compiler_params=pltpu.CompilerParams(
            dimension_semantics=("parallel", "arbitrary"),
            vmem_limit_bytes=48 * 1024 * 1024,
        ),
        name="made_ar_sample",
    )(context, eps, WcT, b1r, W1mT, wmu, wsc, wall, wpart, bmu, bsc)
    return z, mu, sc


# speedup vs baseline: 5.4943x; 1.0192x over previous
import numpy as np
import jax
import jax.numpy as jnp
from jax.experimental import pallas as pl
from jax.experimental.pallas import tpu as pltpu

_LANE = 128
_GROUP = 8
_RSPLIT = 16


def _made_degrees(D, H):
    m0 = np.arange(1, D + 1)
    mh = (np.arange(H) % (D - 1)) + 1
    return m0, mh


def _ar_body(ctx_ref, eps_ref, wct_ref, b1_ref, w1t_ref, wmu_ref, wsc_ref,
             wall_ref, wpart_ref, bmu_ref, bsc_ref,
             z_ref, mu_ref, sc_ref, a_ref, h_ref, zs_ref, part_ref):
    BB = ctx_ref.shape[0]
    D = eps_ref.shape[-1]
    H = a_ref.shape[-1]
    NG = D // _GROUP

    a_ref[...] = jnp.dot(ctx_ref[...], wct_ref[...],
                         preferred_element_type=jnp.float32) + b1_ref[...]

    iota = jax.lax.broadcasted_iota(jnp.int32, (1, D), 1)
    iota16 = jax.lax.broadcasted_iota(jnp.int32, (1, 2 * _GROUP), 1)
    cdims = (((1,), (1,)), ((), ()))

    zs_ref[...] = jnp.zeros((BB, D), jnp.float32)
    for g in range(NG):
        i0 = g * _GROUP
        c0 = g * _LANE
        c1 = min((g + 2) * _LANE, H)

        lz0, lz1 = (g + 1) * _LANE, min((g + 2) * _LANE, H)
        if g >= 1 and lz0 < H:
            a_ref[:, lz0:lz1] = a_ref[:, lz0:lz1] + jax.lax.dot_general(
                zs_ref[...], w1t_ref[:, lz0:lz1], (((1,), (0,)), ((), ())),
                preferred_element_type=jnp.float32)

        has_part = g > 0
        if has_part:
            kf = g * _LANE
            part_ref[...] = jax.lax.dot_general(
                h_ref[:, :kf], wpart_ref[g][:, :kf], cdims,
                preferred_element_type=jnp.float32)

        def step(i, _, has_part=has_part, c0=c0, c1=c1, i0=i0):
            oh = (iota == i).astype(jnp.float32)
            wmu_row = wmu_ref[pl.ds(i, 1), :]
            wsc_row = wsc_ref[pl.ds(i, 1), :]
            w1_row = w1t_ref[pl.ds(i, 1), :]
            j = i - i0
            mpm = (iota16 == j).astype(jnp.float32)
            mpp = (iota16 == j + _GROUP).astype(jnp.float32)
            bmu_i = jnp.sum(bmu_ref[...] * oh, axis=1, keepdims=True)
            bsc_i = jnp.sum(bsc_ref[...] * oh, axis=1, keepdims=True)
            RB = BB // _RSPLIT
            for r0 in range(0, BB, RB):
                r1 = r0 + RB
                win = a_ref[r0:r1, c0:c1]
                h = jnp.maximum(win, 0.0)
                mu = jnp.sum(h * wmu_row[:, c0:c1], axis=1, keepdims=True)
                pre = jnp.sum(h * wsc_row[:, c0:c1], axis=1, keepdims=True)
                if has_part:
                    part = part_ref[r0:r1, :]
                    mu = mu + jnp.sum(part * mpm, axis=1, keepdims=True)
                    pre = pre + jnp.sum(part * mpp, axis=1, keepdims=True)
                mu = mu + bmu_i
                pre = pre + bsc_i
                sc = jax.nn.softplus(pre)
                epsi = jnp.sum(eps_ref[0, r0:r1, :] * oh, axis=1, keepdims=True)
                zi = mu + sc * epsi
                a_ref[r0:r1, c0:c1] = win + zi * w1_row[:, c0:c1]
                zs_ref[r0:r1, :] = zs_ref[r0:r1, :] + zi * oh
            return 0

        jax.lax.fori_loop(i0, i0 + _GROUP, step, 0)

        f1 = min(c0 + _LANE, H)
        h_ref[:, c0:f1] = jnp.maximum(a_ref[:, c0:f1], 0.0)

    z_ref[0] = zs_ref[...]
    out = jax.lax.dot_general(h_ref[...], wall_ref[...], cdims,
                              preferred_element_type=jnp.float32)
    mu_ref[0] = out[:, :D] + bmu_ref[...]
    sc_ref[0] = jax.nn.softplus(out[:, D:] + bsc_ref[...])


def kernel(context, eps, W1, b1, Wc, Wout, bout):
    S, B, D = eps.shape
    H = W1.shape[0]
    CTX = Wc.shape[1]
    m0, mh = _made_degrees(D, H)

    perm = np.argsort(mh, kind="stable")
    mh_s = mh[perm]
    M1 = jnp.asarray((mh_s[:, None] >= m0[None, :]).astype(np.float32))
    Mout = jnp.asarray((m0[:, None] > mh_s[None, :]).astype(np.float32))
    perm_j = jnp.asarray(perm)

    W1p = W1[perm_j]
    Wcp = Wc[perm_j]
    b1p = b1[perm_j]
    Woutp = Wout[:, perm_j]

    W1mT = (W1p * M1).T
    WcT = Wcp.T
    wmu = Woutp[:D] * Mout
    wsc = Woutp[D:] * Mout
    wall = jnp.concatenate([wmu, wsc], axis=0)
    b1r = b1p.reshape(1, H)
    bmu = bout[:D].reshape(1, D)
    bsc = bout[D:].reshape(1, D)

    NG = D // _GROUP
    wpart = jnp.stack([
        jnp.concatenate([wmu[g * _GROUP:(g + 1) * _GROUP],
                         wsc[g * _GROUP:(g + 1) * _GROUP]], axis=0)
        for g in range(NG)])

    NB = 1
    BB = B // NB

    fixed = lambda s, nb: (0, 0)
    fixed3 = lambda s, nb: (0, 0, 0)
    z, mu, sc = pl.pallas_call(
        _ar_body,
        out_shape=[jax.ShapeDtypeStruct((S, B, D), jnp.float32)] * 3,
        grid=(S, NB),
        in_specs=[
            pl.BlockSpec((BB, CTX), lambda s, nb: (nb, 0)),
            pl.BlockSpec((1, BB, D), lambda s, nb: (s, nb, 0)),
            pl.BlockSpec((CTX, H), fixed),
            pl.BlockSpec((1, H), fixed),
            pl.BlockSpec((D, H), fixed),
            pl.BlockSpec((D, H), fixed),
            pl.BlockSpec((D, H), fixed),
            pl.BlockSpec((2 * D, H), fixed),
            pl.BlockSpec((NG, 2 * _GROUP, H), fixed3),
            pl.BlockSpec((1, D), fixed),
            pl.BlockSpec((1, D), fixed),
        ],
        out_specs=[pl.BlockSpec((1, BB, D), lambda s, nb: (s, nb, 0))] * 3,
        scratch_shapes=[pltpu.VMEM((BB, H), jnp.float32),
                        pltpu.VMEM((BB, H), jnp.float32),
                        pltpu.VMEM((BB, D), jnp.float32),
                        pltpu.VMEM((BB, 2 * _GROUP), jnp.float32)],
        compiler_params=pltpu.CompilerParams(
            dimension_semantics=("parallel", "arbitrary"),
            vmem_limit_bytes=48 * 1024 * 1024,
        ),
        name="made_ar_sample",
    )(context, eps, WcT, b1r, W1mT, wmu, wsc, wall, wpart, bmu, bsc)
    return z, mu, sc


# speedup vs baseline: 5.6260x; 1.0240x over previous
import numpy as np
import jax
import jax.numpy as jnp
from jax.experimental import pallas as pl
from jax.experimental.pallas import tpu as pltpu

_LANE = 128
_GROUP = 8
_RSPLIT = 16


def _softplus(x):
    return jnp.maximum(x, 0.0) + jnp.log1p(jnp.exp(-jnp.abs(x)))


def _made_degrees(D, H):
    m0 = np.arange(1, D + 1)
    mh = (np.arange(H) % (D - 1)) + 1
    return m0, mh


def _ar_body(ctx_ref, eps_ref, wct_ref, b1_ref, w1t_ref, wmu_ref, wsc_ref,
             wall_ref, wpart_ref, bmu_ref, bsc_ref,
             z_ref, mu_ref, sc_ref, a_ref, h_ref, zs_ref, part_ref):
    BB = ctx_ref.shape[0]
    D = eps_ref.shape[-1]
    H = a_ref.shape[-1]
    NG = D // _GROUP

    a_ref[...] = jnp.dot(ctx_ref[...], wct_ref[...],
                         preferred_element_type=jnp.float32) + b1_ref[...]

    iota = jax.lax.broadcasted_iota(jnp.int32, (1, D), 1)
    iota16 = jax.lax.broadcasted_iota(jnp.int32, (1, 2 * _GROUP), 1)
    cdims = (((1,), (1,)), ((), ()))

    zs_ref[...] = jnp.zeros((BB, D), jnp.float32)
    for g in range(NG):
        i0 = g * _GROUP
        c0 = g * _LANE
        c1 = min((g + 2) * _LANE, H)

        lz0, lz1 = (g + 1) * _LANE, min((g + 2) * _LANE, H)
        if g >= 1 and lz0 < H:
            a_ref[:, lz0:lz1] = a_ref[:, lz0:lz1] + jax.lax.dot_general(
                zs_ref[...], w1t_ref[:, lz0:lz1], (((1,), (0,)), ((), ())),
                preferred_element_type=jnp.float32)

        has_part = g > 0
        if has_part:
            kf = g * _LANE
            part_ref[...] = jax.lax.dot_general(
                h_ref[:, :kf], wpart_ref[g][:, :kf], cdims,
                preferred_element_type=jnp.float32)

        def step(i, _, has_part=has_part, c0=c0, c1=c1, i0=i0):
            oh = (iota == i).astype(jnp.float32)
            wmu_row = wmu_ref[pl.ds(i, 1), :]
            wsc_row = wsc_ref[pl.ds(i, 1), :]
            w1_row = w1t_ref[pl.ds(i, 1), :]
            j = i - i0
            mpm = (iota16 == j).astype(jnp.float32)
            mpp = (iota16 == j + _GROUP).astype(jnp.float32)
            bmu_i = jnp.sum(bmu_ref[...] * oh, axis=1, keepdims=True)
            bsc_i = jnp.sum(bsc_ref[...] * oh, axis=1, keepdims=True)
            RB = BB // _RSPLIT
            for r0 in range(0, BB, RB):
                r1 = r0 + RB
                win = a_ref[r0:r1, c0:c1]
                h = jnp.maximum(win, 0.0)
                mu = jnp.sum(h * wmu_row[:, c0:c1], axis=1, keepdims=True)
                pre = jnp.sum(h * wsc_row[:, c0:c1], axis=1, keepdims=True)
                if has_part:
                    part = part_ref[r0:r1, :]
                    mu = mu + jnp.sum(part * mpm, axis=1, keepdims=True)
                    pre = pre + jnp.sum(part * mpp, axis=1, keepdims=True)
                mu = mu + bmu_i
                pre = pre + bsc_i
                sc = _softplus(pre)
                epsi = jnp.sum(eps_ref[0, r0:r1, :] * oh, axis=1, keepdims=True)
                zi = mu + sc * epsi
                a_ref[r0:r1, c0:c1] = win + zi * w1_row[:, c0:c1]
                zs_ref[r0:r1, :] = zs_ref[r0:r1, :] + zi * oh
            return 0

        jax.lax.fori_loop(i0, i0 + _GROUP, step, 0)

        f1 = min(c0 + _LANE, H)
        h_ref[:, c0:f1] = jnp.maximum(a_ref[:, c0:f1], 0.0)

    z_ref[0] = zs_ref[...]
    out = jax.lax.dot_general(h_ref[...], wall_ref[...], cdims,
                              preferred_element_type=jnp.float32)
    mu_ref[0] = out[:, :D] + bmu_ref[...]
    sc_ref[0] = _softplus(out[:, D:] + bsc_ref[...])


def kernel(context, eps, W1, b1, Wc, Wout, bout):
    S, B, D = eps.shape
    H = W1.shape[0]
    CTX = Wc.shape[1]
    m0, mh = _made_degrees(D, H)

    perm = np.argsort(mh, kind="stable")
    mh_s = mh[perm]
    M1 = jnp.asarray((mh_s[:, None] >= m0[None, :]).astype(np.float32))
    Mout = jnp.asarray((m0[:, None] > mh_s[None, :]).astype(np.float32))
    perm_j = jnp.asarray(perm)

    W1p = W1[perm_j]
    Wcp = Wc[perm_j]
    b1p = b1[perm_j]
    Woutp = Wout[:, perm_j]

    W1mT = (W1p * M1).T
    WcT = Wcp.T
    wmu = Woutp[:D] * Mout
    wsc = Woutp[D:] * Mout
    wall = jnp.concatenate([wmu, wsc], axis=0)
    b1r = b1p.reshape(1, H)
    bmu = bout[:D].reshape(1, D)
    bsc = bout[D:].reshape(1, D)

    NG = D // _GROUP
    wpart = jnp.stack([
        jnp.concatenate([wmu[g * _GROUP:(g + 1) * _GROUP],
                         wsc[g * _GROUP:(g + 1) * _GROUP]], axis=0)
        for g in range(NG)])

    NB = 1
    BB = B // NB

    fixed = lambda s, nb: (0, 0)
    fixed3 = lambda s, nb: (0, 0, 0)
    z, mu, sc = pl.pallas_call(
        _ar_body,
        out_shape=[jax.ShapeDtypeStruct((S, B, D), jnp.float32)] * 3,
        grid=(S, NB),
        in_specs=[
            pl.BlockSpec((BB, CTX), lambda s, nb: (nb, 0)),
            pl.BlockSpec((1, BB, D), lambda s, nb: (s, nb, 0)),
            pl.BlockSpec((CTX, H), fixed),
            pl.BlockSpec((1, H), fixed),
            pl.BlockSpec((D, H), fixed),
            pl.BlockSpec((D, H), fixed),
            pl.BlockSpec((D, H), fixed),
            pl.BlockSpec((2 * D, H), fixed),
            pl.BlockSpec((NG, 2 * _GROUP, H), fixed3),
            pl.BlockSpec((1, D), fixed),
            pl.BlockSpec((1, D), fixed),
        ],
        out_specs=[pl.BlockSpec((1, BB, D), lambda s, nb: (s, nb, 0))] * 3,
        scratch_shapes=[pltpu.VMEM((BB, H), jnp.float32),
                        pltpu.VMEM((BB, H), jnp.float32),
                        pltpu.VMEM((BB, D), jnp.float32),
                        pltpu.VMEM((BB, 2 * _GROUP), jnp.float32)],
        compiler_params=pltpu.CompilerParams(
            dimension_semantics=("parallel", "arbitrary"),
            vmem_limit_bytes=48 * 1024 * 1024,
        ),
        name="made_ar_sample",
    )(context, eps, WcT, b1r, W1mT, wmu, wsc, wall, wpart, bmu, bsc)
    return z, mu, sc


# speedup vs baseline: 5.9681x; 1.0608x over previous
import numpy as np
import jax
import jax.numpy as jnp
from jax.experimental import pallas as pl
from jax.experimental.pallas import tpu as pltpu

_LANE = 128
_GROUP = 8
_RSPLIT = 16


def _softplus(x):
    return jnp.maximum(x, 0.0) + jnp.log1p(jnp.exp(-jnp.abs(x)))


def _made_degrees(D, H):
    m0 = np.arange(1, D + 1)
    mh = (np.arange(H) % (D - 1)) + 1
    return m0, mh


def _ar_body(ctx_ref, eps_ref, wct_ref, b1_ref, w1t_ref, wmu_ref, wsc_ref,
             wall_ref, wpart_ref, bmu_ref, bsc_ref,
             z_ref, mu_ref, sc_ref, a_ref, h_ref, zs_ref, part_ref):
    BB = ctx_ref.shape[0]
    D = eps_ref.shape[-1]
    H = a_ref.shape[-1]
    NG = D // _GROUP

    a_ref[...] = jnp.dot(ctx_ref[...], wct_ref[...],
                         preferred_element_type=jnp.float32) + b1_ref[...]

    iota = jax.lax.broadcasted_iota(jnp.int32, (1, D), 1)
    iota16 = jax.lax.broadcasted_iota(jnp.int32, (1, 2 * _GROUP), 1)
    iota2 = jax.lax.broadcasted_iota(jnp.int32, (1, 2), 1)
    cdims = (((1,), (1,)), ((), ()))

    zs_ref[...] = jnp.zeros((BB, D), jnp.float32)
    for g in range(NG):
        i0 = g * _GROUP
        c0 = g * _LANE
        c1 = min((g + 2) * _LANE, H)

        lz0, lz1 = (g + 1) * _LANE, min((g + 2) * _LANE, H)
        if g >= 1 and lz0 < H:
            a_ref[:, lz0:lz1] = a_ref[:, lz0:lz1] + jax.lax.dot_general(
                zs_ref[...], w1t_ref[:, lz0:lz1], (((1,), (0,)), ((), ())),
                preferred_element_type=jnp.float32)

        has_part = g > 0
        if has_part:
            kf = g * _LANE
            part_ref[...] = jax.lax.dot_general(
                h_ref[:, :kf], wpart_ref[g][:, :kf], cdims,
                preferred_element_type=jnp.float32)

        def step(i, _, has_part=has_part, c0=c0, c1=c1, i0=i0):
            oh = (iota == i).astype(jnp.float32)
            wmu_row = wmu_ref[pl.ds(i, 1), :]
            wsc_row = wsc_ref[pl.ds(i, 1), :]
            w1_row = w1t_ref[pl.ds(i, 1), :]
            j = i - i0
            mpm = (iota16 == j).astype(jnp.float32)
            mpp = (iota16 == j + _GROUP).astype(jnp.float32)
            pm2 = jnp.concatenate([mpm, mpp], axis=0)
            wpair = jnp.concatenate([wmu_row[:, c0:c1],
                                     wsc_row[:, c0:c1]], axis=0)
            sel0 = (iota2 == 0).astype(jnp.float32)
            sel1 = (iota2 == 1).astype(jnp.float32)
            bmu_i = jnp.sum(bmu_ref[...] * oh, axis=1, keepdims=True)
            bsc_i = jnp.sum(bsc_ref[...] * oh, axis=1, keepdims=True)
            RB = BB // _RSPLIT
            for r0 in range(0, BB, RB):
                r1 = r0 + RB
                win = a_ref[r0:r1, c0:c1]
                h = jnp.maximum(win, 0.0)
                mp = jax.lax.dot_general(h, wpair, cdims,
                                         preferred_element_type=jnp.float32)
                if has_part:
                    mp = mp + jax.lax.dot_general(
                        part_ref[r0:r1, :], pm2, cdims,
                        preferred_element_type=jnp.float32)
                mu = jnp.sum(mp * sel0, axis=1, keepdims=True) + bmu_i
                pre = jnp.sum(mp * sel1, axis=1, keepdims=True) + bsc_i
                sc = _softplus(pre)
                epsi = jnp.sum(eps_ref[0, r0:r1, :] * oh, axis=1, keepdims=True)
                zi = mu + sc * epsi
                a_ref[r0:r1, c0:c1] = win + zi * w1_row[:, c0:c1]
                zs_ref[r0:r1, :] = zs_ref[r0:r1, :] + zi * oh
            return 0

        jax.lax.fori_loop(i0, i0 + _GROUP, step, 0)

        f1 = min(c0 + _LANE, H)
        h_ref[:, c0:f1] = jnp.maximum(a_ref[:, c0:f1], 0.0)

    z_ref[0] = zs_ref[...]
    out = jax.lax.dot_general(h_ref[...], wall_ref[...], cdims,
                              preferred_element_type=jnp.float32)
    mu_ref[0] = out[:, :D] + bmu_ref[...]
    sc_ref[0] = _softplus(out[:, D:] + bsc_ref[...])


def kernel(context, eps, W1, b1, Wc, Wout, bout):
    S, B, D = eps.shape
    H = W1.shape[0]
    CTX = Wc.shape[1]
    m0, mh = _made_degrees(D, H)

    perm = np.argsort(mh, kind="stable")
    mh_s = mh[perm]
    M1 = jnp.asarray((mh_s[:, None] >= m0[None, :]).astype(np.float32))
    Mout = jnp.asarray((m0[:, None] > mh_s[None, :]).astype(np.float32))
    perm_j = jnp.asarray(perm)

    W1p = W1[perm_j]
    Wcp = Wc[perm_j]
    b1p = b1[perm_j]
    Woutp = Wout[:, perm_j]

    W1mT = (W1p * M1).T
    WcT = Wcp.T
    wmu = Woutp[:D] * Mout
    wsc = Woutp[D:] * Mout
    wall = jnp.concatenate([wmu, wsc], axis=0)
    b1r = b1p.reshape(1, H)
    bmu = bout[:D].reshape(1, D)
    bsc = bout[D:].reshape(1, D)

    NG = D // _GROUP
    wpart = jnp.stack([
        jnp.concatenate([wmu[g * _GROUP:(g + 1) * _GROUP],
                         wsc[g * _GROUP:(g + 1) * _GROUP]], axis=0)
        for g in range(NG)])

    NB = 1
    BB = B // NB

    fixed = lambda s, nb: (0, 0)
    fixed3 = lambda s, nb: (0, 0, 0)
    z, mu, sc = pl.pallas_call(
        _ar_body,
        out_shape=[jax.ShapeDtypeStruct((S, B, D), jnp.float32)] * 3,
        grid=(S, NB),
        in_specs=[
            pl.BlockSpec((BB, CTX), lambda s, nb: (nb, 0)),
            pl.BlockSpec((1, BB, D), lambda s, nb: (s, nb, 0)),
            pl.BlockSpec((CTX, H), fixed),
            pl.BlockSpec((1, H), fixed),
            pl.BlockSpec((D, H), fixed),
            pl.BlockSpec((D, H), fixed),
            pl.BlockSpec((D, H), fixed),
            pl.BlockSpec((2 * D, H), fixed),
            pl.BlockSpec((NG, 2 * _GROUP, H), fixed3),
            pl.BlockSpec((1, D), fixed),
            pl.BlockSpec((1, D), fixed),
        ],
        out_specs=[pl.BlockSpec((1, BB, D), lambda s, nb: (s, nb, 0))] * 3,
        scratch_shapes=[pltpu.VMEM((BB, H), jnp.float32),
                        pltpu.VMEM((BB, H), jnp.float32),
                        pltpu.VMEM((BB, D), jnp.float32),
                        pltpu.VMEM((BB, 2 * _GROUP), jnp.float32)],
        compiler_params=pltpu.CompilerParams(
            dimension_semantics=("parallel", "arbitrary"),
            vmem_limit_bytes=48 * 1024 * 1024,
        ),
        name="made_ar_sample",
    )(context, eps, WcT, b1r, W1mT, wmu, wsc, wall, wpart, bmu, bsc)
    return z, mu, sc


# speedup vs baseline: 5.9896x; 1.0036x over previous
import numpy as np
import jax
import jax.numpy as jnp
from jax.experimental import pallas as pl
from jax.experimental.pallas import tpu as pltpu

_LANE = 128
_GROUP = 8
_RSPLIT = 32


def _softplus(x):
    return jnp.maximum(x, 0.0) + jnp.log1p(jnp.exp(-jnp.abs(x)))


def _made_degrees(D, H):
    m0 = np.arange(1, D + 1)
    mh = (np.arange(H) % (D - 1)) + 1
    return m0, mh


def _ar_body(ctx_ref, eps_ref, wct_ref, b1_ref, w1t_ref, wmu_ref, wsc_ref,
             wall_ref, wpart_ref, bmu_ref, bsc_ref,
             z_ref, mu_ref, sc_ref, a_ref, h_ref, zs_ref, part_ref):
    BB = ctx_ref.shape[0]
    D = eps_ref.shape[-1]
    H = a_ref.shape[-1]
    NG = D // _GROUP

    a_ref[...] = jnp.dot(ctx_ref[...], wct_ref[...],
                         preferred_element_type=jnp.float32) + b1_ref[...]

    iota = jax.lax.broadcasted_iota(jnp.int32, (1, D), 1)
    iota16 = jax.lax.broadcasted_iota(jnp.int32, (1, 2 * _GROUP), 1)
    iota2 = jax.lax.broadcasted_iota(jnp.int32, (1, 2), 1)
    cdims = (((1,), (1,)), ((), ()))

    zs_ref[...] = jnp.zeros((BB, D), jnp.float32)
    for g in range(NG):
        i0 = g * _GROUP
        c0 = g * _LANE
        c1 = min((g + 2) * _LANE, H)

        lz0, lz1 = (g + 1) * _LANE, min((g + 2) * _LANE, H)
        if g >= 1 and lz0 < H:
            a_ref[:, lz0:lz1] = a_ref[:, lz0:lz1] + jax.lax.dot_general(
                zs_ref[...], w1t_ref[:, lz0:lz1], (((1,), (0,)), ((), ())),
                preferred_element_type=jnp.float32)

        has_part = g > 0
        if has_part:
            kf = g * _LANE
            part_ref[...] = jax.lax.dot_general(
                h_ref[:, :kf], wpart_ref[g][:, :kf], cdims,
                preferred_element_type=jnp.float32)

        def step(i, _, has_part=has_part, c0=c0, c1=c1, i0=i0):
            oh = (iota == i).astype(jnp.float32)
            wmu_row = wmu_ref[pl.ds(i, 1), :]
            wsc_row = wsc_ref[pl.ds(i, 1), :]
            w1_row = w1t_ref[pl.ds(i, 1), :]
            j = i - i0
            mpm = (iota16 == j).astype(jnp.float32)
            mpp = (iota16 == j + _GROUP).astype(jnp.float32)
            pm2 = jnp.concatenate([mpm, mpp], axis=0)
            wpair = jnp.concatenate([wmu_row[:, c0:c1],
                                     wsc_row[:, c0:c1]], axis=0)
            sel0 = (iota2 == 0).astype(jnp.float32)
            sel1 = (iota2 == 1).astype(jnp.float32)
            bmu_i = jnp.sum(bmu_ref[...] * oh, axis=1, keepdims=True)
            bsc_i = jnp.sum(bsc_ref[...] * oh, axis=1, keepdims=True)
            RB = BB // _RSPLIT
            for r0 in range(0, BB, RB):
                r1 = r0 + RB
                win = a_ref[r0:r1, c0:c1]
                h = jnp.maximum(win, 0.0)
                mp = jax.lax.dot_general(h, wpair, cdims,
                                         preferred_element_type=jnp.float32)
                if has_part:
                    mp = mp + jax.lax.dot_general(
                        part_ref[r0:r1, :], pm2, cdims,
                        preferred_element_type=jnp.float32)
                mu = jnp.sum(mp * sel0, axis=1, keepdims=True) + bmu_i
                pre = jnp.sum(mp * sel1, axis=1, keepdims=True) + bsc_i
                sc = _softplus(pre)
                epsi = jnp.sum(eps_ref[0, r0:r1, :] * oh, axis=1, keepdims=True)
                zi = mu + sc * epsi
                a_ref[r0:r1, c0:c1] = win + zi * w1_row[:, c0:c1]
                zs_ref[r0:r1, :] = zs_ref[r0:r1, :] + zi * oh
            return 0

        jax.lax.fori_loop(i0, i0 + _GROUP, step, 0)

        f1 = min(c0 + _LANE, H)
        h_ref[:, c0:f1] = jnp.maximum(a_ref[:, c0:f1], 0.0)

    z_ref[0] = zs_ref[...]
    out = jax.lax.dot_general(h_ref[...], wall_ref[...], cdims,
                              preferred_element_type=jnp.float32)
    mu_ref[0] = out[:, :D] + bmu_ref[...]
    sc_ref[0] = _softplus(out[:, D:] + bsc_ref[...])


def kernel(context, eps, W1, b1, Wc, Wout, bout):
    S, B, D = eps.shape
    H = W1.shape[0]
    CTX = Wc.shape[1]
    m0, mh = _made_degrees(D, H)

    perm = np.argsort(mh, kind="stable")
    mh_s = mh[perm]
    M1 = jnp.asarray((mh_s[:, None] >= m0[None, :]).astype(np.float32))
    Mout = jnp.asarray((m0[:, None] > mh_s[None, :]).astype(np.float32))
    perm_j = jnp.asarray(perm)

    W1p = W1[perm_j]
    Wcp = Wc[perm_j]
    b1p = b1[perm_j]
    Woutp = Wout[:, perm_j]

    W1mT = (W1p * M1).T
    WcT = Wcp.T
    wmu = Woutp[:D] * Mout
    wsc = Woutp[D:] * Mout
    wall = jnp.concatenate([wmu, wsc], axis=0)
    b1r = b1p.reshape(1, H)
    bmu = bout[:D].reshape(1, D)
    bsc = bout[D:].reshape(1, D)

    NG = D // _GROUP
    wpart = jnp.stack([
        jnp.concatenate([wmu[g * _GROUP:(g + 1) * _GROUP],
                         wsc[g * _GROUP:(g + 1) * _GROUP]], axis=0)
        for g in range(NG)])

    NB = 1
    BB = B // NB

    fixed = lambda s, nb: (0, 0)
    fixed3 = lambda s, nb: (0, 0, 0)
    z, mu, sc = pl.pallas_call(
        _ar_body,
        out_shape=[jax.ShapeDtypeStruct((S, B, D), jnp.float32)] * 3,
        grid=(S, NB),
        in_specs=[
            pl.BlockSpec((BB, CTX), lambda s, nb: (nb, 0)),
            pl.BlockSpec((1, BB, D), lambda s, nb: (s, nb, 0)),
            pl.BlockSpec((CTX, H), fixed),
            pl.BlockSpec((1, H), fixed),
            pl.BlockSpec((D, H), fixed),
            pl.BlockSpec((D, H), fixed),
            pl.BlockSpec((D, H), fixed),
            pl.BlockSpec((2 * D, H), fixed),
            pl.BlockSpec((NG, 2 * _GROUP, H), fixed3),
            pl.BlockSpec((1, D), fixed),
            pl.BlockSpec((1, D), fixed),
        ],
        out_specs=[pl.BlockSpec((1, BB, D), lambda s, nb: (s, nb, 0))] * 3,
        scratch_shapes=[pltpu.VMEM((BB, H), jnp.float32),
                        pltpu.VMEM((BB, H), jnp.float32),
                        pltpu.VMEM((BB, D), jnp.float32),
                        pltpu.VMEM((BB, 2 * _GROUP), jnp.float32)],
        compiler_params=pltpu.CompilerParams(
            dimension_semantics=("parallel", "arbitrary"),
            vmem_limit_bytes=48 * 1024 * 1024,
        ),
        name="made_ar_sample",
    )(context, eps, WcT, b1r, W1mT, wmu, wsc, wall, wpart, bmu, bsc)
    return z, mu, sc
